# baseline (speedup 1.0000x reference)
import functools

import jax
import jax.numpy as jnp
from jax import lax
from jax.experimental import pallas as pl
from jax.experimental.pallas import tpu as pltpu
from jax.experimental.pallas import tpu_sc as plsc

Q = 8192
T = 8192
D = 1024
BQ = 4096
BT = 256
NQ = Q // BQ
NT = T // BT
S = NT // 2
CR = 256
NCR = BQ // CR
LW = 128

_DN = (((1,), (1,)), ((), ()))


def _phase(s_ref, lm_ref, li_ref, dsts, t_refs, srcs, ns_ref, row_base,
           nt_refs, col_blocks, valid):
    for r in range(NCR):
        sl = pl.ds(r * CR, CR)
        for dst, t_ref in zip(dsts, t_refs):
            dst[sl, :] = lax.dot_general(
                s_ref[sl, :], t_ref[...], _DN,
                preferred_element_type=jnp.float32)
        for src, nt_ref, cb in zip(srcs, nt_refs, col_blocks):
            ns = ns_ref[pl.ds(row_base + r * CR, CR), :]
            for c in range(BT // LW):
                cs = pl.ds(c * LW, LW)
                d = 1.0 - src[sl, cs] / (ns * nt_ref[:, cs])
                col = (cb * BT + c * LW
                       + lax.broadcasted_iota(jnp.int32, (CR, LW), 1))
                upd = jnp.logical_and(d < lm_ref[sl, :], valid)
                lm_ref[sl, :] = jnp.where(upd, d, lm_ref[sl, :])
                li_ref[sl, :] = jnp.where(upd, col, li_ref[sl, :])


def _argmin_body(s_ref, ta_ref, tb_ref,
                 ns_ref, ntp_ref, nta_ref,
                 idx_ref, a_buf, b_buf, lm_ref, li_ref):
    g = pl.program_id(0)
    q = g % S
    pe = (g - 1) % S
    glast = pl.num_programs(0) - 1
    ip = jnp.maximum((g - 1) // S, 0)
    iq = jnp.minimum(g // S, NQ - 1)

    _phase(s_ref, lm_ref, li_ref,
           (a_buf,), (ta_ref,),
           (b_buf,), ns_ref, ip * BQ, (ntp_ref,),
           (2 * pe + 1,), g > 0)

    @pl.when(jnp.logical_and(g > 0, q == 0))
    def _emit():
        lm = lm_ref[...]
        m = jnp.min(lm, axis=1, keepdims=True)
        idx_ref[pl.ds(ip * BQ, BQ), :] = jnp.min(
            jnp.where(lm == m, li_ref[...], T), axis=1, keepdims=True)

    @pl.when(q == 0)
    def _init():
        lm_ref[...] = jnp.full((BQ, LW), jnp.inf, jnp.float32)
        li_ref[...] = jnp.zeros((BQ, LW), jnp.int32)

    _phase(s_ref, lm_ref, li_ref,
           (b_buf,), (tb_ref,),
           (a_buf,), ns_ref, iq * BQ, (nta_ref,),
           (2 * q,), g < glast)


def _argmin_indices(source, target, ns_col, nt_row, interpret=False):
    grid = (NQ * S + 1,)
    return pl.pallas_call(
        _argmin_body,
        grid=grid,
        in_specs=[
            pl.BlockSpec((BQ, D), lambda g: (jnp.minimum(g // S, NQ - 1), 0)),
            pl.BlockSpec((BT, D), lambda g: (2 * (g % S), 0)),
            pl.BlockSpec((BT, D), lambda g: (2 * (g % S) + 1, 0)),
            pl.BlockSpec((Q, 1), lambda g: (0, 0)),
            pl.BlockSpec((1, BT), lambda g: (0, 2 * ((g - 1) % S) + 1)),
            pl.BlockSpec((1, BT), lambda g: (0, 2 * (g % S))),
        ],
        out_specs=pl.BlockSpec((Q, 1), lambda g: (0, 0)),
        out_shape=jax.ShapeDtypeStruct((Q, 1), jnp.int32),
        scratch_shapes=[
            pltpu.VMEM((BQ, BT), jnp.float32),
            pltpu.VMEM((BQ, BT), jnp.float32),
            pltpu.VMEM((BQ, LW), jnp.float32),
            pltpu.VMEM((BQ, LW), jnp.int32),
        ],
        compiler_params=pltpu.CompilerParams(
            dimension_semantics=("arbitrary",)),
        interpret=interpret,
    )(source, target, target, ns_col, nt_row, nt_row)


_NC = 2
_NS = 16
_NW = _NC * _NS
_BPW = Q // _NW
_CH = 64
_NCH = _BPW // _CH


def _gather_body(table_hbm, idx_hbm, out_hbm, idx_v, rows_v, sem):
    wid = lax.axis_index("s") * _NC + lax.axis_index("c")
    base = wid * _BPW
    for c in range(_NCH):
        off = base + c * _CH
        pltpu.sync_copy(idx_hbm.at[pl.ds(off, _CH)], idx_v)
        pltpu.async_copy(table_hbm.at[idx_v], rows_v, sem).wait()
        pltpu.sync_copy(rows_v, out_hbm.at[pl.ds(off, _CH)])


def _sc_gather(table, idx):
    k = functools.partial(
        pl.kernel,
        mesh=plsc.VectorSubcoreMesh(
            core_axis_name="c", subcore_axis_name="s",
            num_cores=_NC, num_subcores=_NS),
        out_type=jax.ShapeDtypeStruct((Q, D), jnp.float32),
        scratch_types=[
            pltpu.VMEM((_CH,), jnp.int32),
            pltpu.VMEM((_CH, D), jnp.float32),
            pltpu.SemaphoreType.DMA,
        ],
    )(_gather_body)
    return k(table, idx)


def kernel(source_features, target_features):
    source_norms = jnp.linalg.norm(source_features, axis=-1)
    matching_norms = jnp.linalg.norm(target_features, axis=-1)
    idx = _argmin_indices(
        source_features, target_features,
        source_norms.reshape(Q, 1), matching_norms.reshape(1, T))
    idx = idx.reshape(Q)
    return _sc_gather(target_features, idx)


# speedup vs baseline: 1.0445x; 1.0445x over previous
import functools

import jax
import jax.numpy as jnp
from jax import lax
from jax.experimental import pallas as pl
from jax.experimental.pallas import tpu as pltpu
from jax.experimental.pallas import tpu_sc as plsc

Q = 8192
T = 8192
D = 1024
BQ = 2048
BT = 256
NQ = Q // BQ
NT = T // BT
S = NT // 2
CR = 256
NCR = BQ // CR
LW = 128

_DN = (((1,), (1,)), ((), ()))


def _phase(s_ref, lm_ref, li_ref, dst, t_ref, src, ns_ref, row_base,
           nt_ref, cb, valid, first):
    for r in range(NCR):
        sl = pl.ds(r * CR, CR)
        dst[sl, :] = lax.dot_general(
            s_ref[sl, :], t_ref[...], _DN,
            preferred_element_type=jnp.float32)
        ns = ns_ref[pl.ds(row_base + r * CR, CR), :]
        for c in range(BT // LW):
            cs = pl.ds(c * LW, LW)
            d = 1.0 - src[sl, cs] / (ns * nt_ref[:, cs])
            col = (cb * BT + c * LW
                   + lax.broadcasted_iota(jnp.int32, (CR, LW), 1))
            upd = jnp.logical_and(d < lm_ref[sl, :], valid)
            if c == 0:
                upd = jnp.logical_or(upd, first)
            lm_ref[sl, :] = jnp.where(upd, d, lm_ref[sl, :])
            li_ref[sl, :] = jnp.where(upd, col, li_ref[sl, :])


def _argmin_body(s_ref, ta_ref, tb_ref, ns_ref, ntp_ref, nta_ref,
                 lm_ref, li_ref, a_buf, b_buf):
    g = pl.program_id(0)
    q = g % (S + 1)
    row_base = (g // (S + 1)) * BQ

    _phase(s_ref, lm_ref, li_ref, a_buf, ta_ref, b_buf,
           ns_ref, row_base, ntp_ref, 2 * q - 1,
           q > 0, False)

    _phase(s_ref, lm_ref, li_ref, b_buf, tb_ref, a_buf,
           ns_ref, row_base, nta_ref, 2 * q,
           q < S, q == 0)


def _argmin_state(source, target, ns_col, nt_row, interpret=False):
    sp1 = S + 1
    grid = (NQ * sp1,)
    return pl.pallas_call(
        _argmin_body,
        grid=grid,
        in_specs=[
            pl.BlockSpec((BQ, D), lambda g: (g // (S + 1), 0)),
            pl.BlockSpec((BT, D),
                         lambda g: (2 * jnp.minimum(g % (S + 1), S - 1), 0)),
            pl.BlockSpec((BT, D),
                         lambda g: (2 * jnp.minimum(g % (S + 1), S - 1) + 1, 0)),
            pl.BlockSpec((Q, 1), lambda g: (0, 0)),
            pl.BlockSpec((1, BT),
                         lambda g: (0, jnp.maximum(2 * (g % (S + 1)) - 1, 0))),
            pl.BlockSpec((1, BT),
                         lambda g: (0, jnp.minimum(2 * (g % (S + 1)), NT - 1))),
        ],
        out_specs=[
            pl.BlockSpec((BQ, LW), lambda g: (g // (S + 1), 0)),
            pl.BlockSpec((BQ, LW), lambda g: (g // (S + 1), 0)),
        ],
        out_shape=[
            jax.ShapeDtypeStruct((Q, LW), jnp.float32),
            jax.ShapeDtypeStruct((Q, LW), jnp.int32),
        ],
        scratch_shapes=[
            pltpu.VMEM((BQ, BT), jnp.float32),
            pltpu.VMEM((BQ, BT), jnp.float32),
        ],
        compiler_params=pltpu.CompilerParams(
            dimension_semantics=("arbitrary",)),
        interpret=interpret,
    )(source, target, target, ns_col, nt_row, nt_row)


def _finalize_body(lm_ref, li_ref, idx_ref):
    lm = lm_ref[...]
    m = jnp.min(lm, axis=1, keepdims=True)
    idx_ref[...] = jnp.min(
        jnp.where(lm == m, li_ref[...], T), axis=1, keepdims=True)


def _finalize(lm, li, interpret=False):
    return pl.pallas_call(
        _finalize_body,
        grid=(NQ,),
        in_specs=[
            pl.BlockSpec((BQ, LW), lambda i: (i, 0)),
            pl.BlockSpec((BQ, LW), lambda i: (i, 0)),
        ],
        out_specs=pl.BlockSpec((BQ, 1), lambda i: (i, 0)),
        out_shape=jax.ShapeDtypeStruct((Q, 1), jnp.int32),
        interpret=interpret,
    )(lm, li)


_NC = 2
_NS = 16
_NW = _NC * _NS
_BPW = Q // _NW
_CH = 64
_NCH = _BPW // _CH


def _gather_body(table_hbm, idx_hbm, out_hbm, idx_v, rows_v, sem):
    wid = lax.axis_index("s") * _NC + lax.axis_index("c")
    base = wid * _BPW
    for c in range(_NCH):
        off = base + c * _CH
        pltpu.sync_copy(idx_hbm.at[pl.ds(off, _CH)], idx_v)
        pltpu.async_copy(table_hbm.at[idx_v], rows_v, sem).wait()
        pltpu.sync_copy(rows_v, out_hbm.at[pl.ds(off, _CH)])


def _sc_gather(table, idx):
    k = functools.partial(
        pl.kernel,
        mesh=plsc.VectorSubcoreMesh(
            core_axis_name="c", subcore_axis_name="s",
            num_cores=_NC, num_subcores=_NS),
        out_type=jax.ShapeDtypeStruct((Q, D), jnp.float32),
        scratch_types=[
            pltpu.VMEM((_CH,), jnp.int32),
            pltpu.VMEM((_CH, D), jnp.float32),
            pltpu.SemaphoreType.DMA,
        ],
    )(_gather_body)
    return k(table, idx)


def kernel(source_features, target_features):
    source_norms = jnp.linalg.norm(source_features, axis=-1)
    matching_norms = jnp.linalg.norm(target_features, axis=-1)
    lm, li = _argmin_state(
        source_features, target_features,
        source_norms.reshape(Q, 1), matching_norms.reshape(1, T))
    idx = _finalize(lm, li).reshape(Q)
    return _sc_gather(target_features, idx)


# speedup vs baseline: 1.1133x; 1.0659x over previous
import functools

import jax
import jax.numpy as jnp
from jax import lax
from jax.experimental import pallas as pl
from jax.experimental.pallas import tpu as pltpu
from jax.experimental.pallas import tpu_sc as plsc

Q = 8192
T = 8192
D = 1024
BQ = 4096
BT = 256
NQ = Q // BQ
NT = T // BT
S = NT // 2
CR = 256
NCR = BQ // CR
LW = 128

_DN = (((1,), (1,)), ((), ()))


def _phase(s_ref, lm_ref, li_ref, dst, t_ref, src, ns_ref, row_base,
           nt_ref, cb, valid, first):
    for r in range(NCR):
        sl = pl.ds(r * CR, CR)
        dst[sl, :] = lax.dot_general(
            s_ref[sl, :], t_ref[...], _DN,
            preferred_element_type=jnp.float32)
        ns = ns_ref[pl.ds(row_base + r * CR, CR), :]
        for c in range(BT // LW):
            cs = pl.ds(c * LW, LW)
            d = 1.0 - src[sl, cs] / (ns * nt_ref[:, cs])
            col = (cb * BT + c * LW
                   + lax.broadcasted_iota(jnp.int32, (CR, LW), 1))
            upd = jnp.logical_and(d < lm_ref[sl, :], valid)
            if c == 0:
                upd = jnp.logical_or(upd, first)
            lm_ref[sl, :] = jnp.where(upd, d, lm_ref[sl, :])
            li_ref[sl, :] = jnp.where(upd, col, li_ref[sl, :])


def _argmin_body(s_ref, ta_ref, tb_ref, ns_ref, ntp_ref, nta_ref,
                 lm_ref, li_ref, a_buf, b_buf):
    g = pl.program_id(0)
    q = g % (S + 1)
    row_base = (g // (S + 1)) * BQ

    _phase(s_ref, lm_ref, li_ref, a_buf, ta_ref, b_buf,
           ns_ref, row_base, ntp_ref, 2 * q - 1,
           q > 0, False)

    _phase(s_ref, lm_ref, li_ref, b_buf, tb_ref, a_buf,
           ns_ref, row_base, nta_ref, 2 * q,
           q < S, q == 0)


def _argmin_state(source, target, ns_col, nt_row, interpret=False):
    sp1 = S + 1
    grid = (NQ * sp1,)
    return pl.pallas_call(
        _argmin_body,
        grid=grid,
        in_specs=[
            pl.BlockSpec((BQ, D), lambda g: (g // (S + 1), 0)),
            pl.BlockSpec((BT, D),
                         lambda g: (2 * jnp.minimum(g % (S + 1), S - 1), 0)),
            pl.BlockSpec((BT, D),
                         lambda g: (2 * jnp.minimum(g % (S + 1), S - 1) + 1, 0)),
            pl.BlockSpec((Q, 1), lambda g: (0, 0)),
            pl.BlockSpec((1, BT),
                         lambda g: (0, jnp.maximum(2 * (g % (S + 1)) - 1, 0))),
            pl.BlockSpec((1, BT),
                         lambda g: (0, jnp.minimum(2 * (g % (S + 1)), NT - 1))),
        ],
        out_specs=[
            pl.BlockSpec((BQ, LW), lambda g: (g // (S + 1), 0)),
            pl.BlockSpec((BQ, LW), lambda g: (g // (S + 1), 0)),
        ],
        out_shape=[
            jax.ShapeDtypeStruct((Q, LW), jnp.float32),
            jax.ShapeDtypeStruct((Q, LW), jnp.int32),
        ],
        scratch_shapes=[
            pltpu.VMEM((BQ, BT), jnp.float32),
            pltpu.VMEM((BQ, BT), jnp.float32),
        ],
        compiler_params=pltpu.CompilerParams(
            dimension_semantics=("arbitrary",)),
        interpret=interpret,
    )(source, target, target, ns_col, nt_row, nt_row)


def _finalize_body(lm_ref, li_ref, idx_ref):
    lm = lm_ref[...]
    m = jnp.min(lm, axis=1, keepdims=True)
    idx_ref[...] = jnp.min(
        jnp.where(lm == m, li_ref[...], T), axis=1, keepdims=True)


def _finalize(lm, li, interpret=False):
    return pl.pallas_call(
        _finalize_body,
        grid=(NQ,),
        in_specs=[
            pl.BlockSpec((BQ, LW), lambda i: (i, 0)),
            pl.BlockSpec((BQ, LW), lambda i: (i, 0)),
        ],
        out_specs=pl.BlockSpec((BQ, 1), lambda i: (i, 0)),
        out_shape=jax.ShapeDtypeStruct((Q, 1), jnp.int32),
        interpret=interpret,
    )(lm, li)


_NC = 2
_NS = 16
_NW = _NC * _NS
_BPW = Q // _NW
_CH = 64
_NCH = _BPW // _CH


def _gather_body(table_hbm, idx_hbm, out_hbm, idx_v, rows_v, sem):
    wid = lax.axis_index("s") * _NC + lax.axis_index("c")
    base = wid * _BPW
    for c in range(_NCH):
        off = base + c * _CH
        pltpu.sync_copy(idx_hbm.at[pl.ds(off, _CH)], idx_v)
        pltpu.async_copy(table_hbm.at[idx_v], rows_v, sem).wait()
        pltpu.sync_copy(rows_v, out_hbm.at[pl.ds(off, _CH)])


def _sc_gather(table, idx):
    k = functools.partial(
        pl.kernel,
        mesh=plsc.VectorSubcoreMesh(
            core_axis_name="c", subcore_axis_name="s",
            num_cores=_NC, num_subcores=_NS),
        out_type=jax.ShapeDtypeStruct((Q, D), jnp.float32),
        scratch_types=[
            pltpu.VMEM((_CH,), jnp.int32),
            pltpu.VMEM((_CH, D), jnp.float32),
            pltpu.SemaphoreType.DMA,
        ],
    )(_gather_body)
    return k(table, idx)


def kernel(source_features, target_features):
    source_norms = jnp.linalg.norm(source_features, axis=-1)
    matching_norms = jnp.linalg.norm(target_features, axis=-1)
    lm, li = _argmin_state(
        source_features, target_features,
        source_norms.reshape(Q, 1), matching_norms.reshape(1, T))
    idx = _finalize(lm, li).reshape(Q)
    return _sc_gather(target_features, idx)
